# baseline (speedup 1.0000x reference)
import functools

import jax
import jax.numpy as jnp
from jax import lax
from jax.experimental import pallas as pl
from jax.experimental.pallas import tpu as pltpu
from jax.experimental.pallas import tpu_sc as plsc

T = 8192
D = 768
F = 128
E = 8
TB = 1024
EF = E * F
PW = 128

_INFO = plsc.get_sparse_core_info()
NC = _INFO.num_cores
NS = _INFO.num_subcores
L = _INFO.num_lanes
NW = NC * NS
CHUNK = T // NW


def _logits_kernel(x_ref, wr_ref, br_ref, lt_ref):
    lg = jnp.dot(x_ref[...], wr_ref[...], preferred_element_type=jnp.float32)
    lt_ref[...] = (lg + br_ref[...]).T


def _route_sc_body(lt_hbm, wt_hbm, imp_hbm, lg_v, wv, impv):
    wid = lax.axis_index("s") * NC + lax.axis_index("c")
    base = wid * CHUNK
    pltpu.sync_copy(lt_hbm.at[:, pl.ds(base, CHUNK)], lg_v)

    acc = [jnp.zeros((L,), jnp.float32) for _ in range(E)]
    for g in range(CHUNK // L):
        sl = pl.ds(g * L, L)
        l = [lg_v[e, sl] for e in range(E)]
        m = l[0]
        for e in range(1, E):
            m = jnp.maximum(m, l[e])
        ex = [jnp.exp(l[e] - m) for e in range(E)]
        s = ex[0]
        for e in range(1, E):
            s = s + ex[e]
        sc = [ex[e] / s for e in range(E)]

        v1 = sc[0]
        for e in range(1, E):
            v1 = jnp.maximum(v1, sc[e])
        big = jnp.full((L,), E, jnp.int32)
        idx1 = jnp.where(sc[0] == v1, jnp.full((L,), 0, jnp.int32), big)
        for e in range(1, E):
            idx1 = jnp.minimum(
                idx1, jnp.where(sc[e] == v1, jnp.full((L,), e, jnp.int32), big))
        ninf = jnp.full((L,), -jnp.inf, jnp.float32)
        s2 = [jnp.where(idx1 == e, ninf, sc[e]) for e in range(E)]
        v2 = s2[0]
        for e in range(1, E):
            v2 = jnp.maximum(v2, s2[e])
        idx2 = jnp.where(s2[0] == v2, jnp.full((L,), 0, jnp.int32), big)
        for e in range(1, E):
            idx2 = jnp.minimum(
                idx2, jnp.where(s2[e] == v2, jnp.full((L,), e, jnp.int32), big))
        zero = jnp.zeros((L,), jnp.float32)
        for e in range(E):
            w_e = jnp.where((idx1 == e) | (idx2 == e), sc[e], zero)
            wv[e, sl] = w_e
            acc[e] = acc[e] + w_e

    for e in range(E):
        impv[e, :] = acc[e]

    pltpu.sync_copy(wv, wt_hbm.at[:, pl.ds(base, CHUNK)])
    pltpu.sync_copy(impv, imp_hbm.at[wid])


_route_sc = functools.partial(
    pl.kernel,
    mesh=plsc.VectorSubcoreMesh(core_axis_name="c", subcore_axis_name="s"),
    out_type=[
        jax.ShapeDtypeStruct((E, T), jnp.float32),
        jax.ShapeDtypeStruct((NW, E, L), jnp.float32),
    ],
    scratch_types=[
        pltpu.VMEM((E, CHUNK), jnp.float32),
        pltpu.VMEM((E, CHUNK), jnp.float32),
        pltpu.VMEM((E, L), jnp.float32),
    ],
)(_route_sc_body)


def _moe_kernel(x_ref, wt_ref, imp_ref, w1_ref, b1_ref, w2_ref, b2_ref,
                sel_ref, out_ref, loss_ref, w1c_ref, w2a_ref, *, num_tiles):
    i = pl.program_id(0)

    @pl.when(i == 0)
    def _stage():
        for e_i in range(E):
            w1c_ref[:, e_i * F:(e_i + 1) * F] = (
                w1_ref[e_i].astype(jnp.bfloat16))
        w2a_ref[0:EF, :] = w2_ref[...].astype(jnp.bfloat16)
        w2a_ref[EF:EF + E, :] = b2_ref[...].astype(jnp.bfloat16)
        w2a_ref[EF + E:, :] = jnp.zeros((PW - E, D), jnp.bfloat16)

    x = x_ref[...]
    w = wt_ref[...].T

    xb = x.astype(jnp.bfloat16)
    h = jnp.dot(xb, w1c_ref[...], preferred_element_type=jnp.float32)
    h = jnp.maximum(h + b1_ref[...], 0.0)

    wpad_f = jnp.pad(w, ((0, 0), (0, PW - E)))
    wexp = jnp.dot(wpad_f, sel_ref[...], preferred_element_type=jnp.float32)
    hw = (h * wexp).astype(jnp.bfloat16)
    hcat = jnp.concatenate([hw, wpad_f.astype(jnp.bfloat16)], axis=-1)
    out_ref[...] = jnp.dot(hcat, w2a_ref[...],
                           preferred_element_type=jnp.float32)

    @pl.when(i == num_tiles - 1)
    def _loss():
        s = [jnp.sum(imp_ref[:, e, :]) for e in range(E)]
        mean = sum(s) / E
        var = sum((s_e - mean) ** 2 for s_e in s) / (E - 1)
        loss_ref[...] = jnp.full((1, 1), var / (mean * mean + 1e-9),
                                 jnp.float32)


def kernel(x, Wr, br, W1, b1, W2, b2):
    num_tiles = T // TB
    lt = pl.pallas_call(
        _logits_kernel,
        grid=(num_tiles,),
        in_specs=[
            pl.BlockSpec((TB, D), lambda i: (i, 0)),
            pl.BlockSpec((D, E), lambda i: (0, 0)),
            pl.BlockSpec((1, E), lambda i: (0, 0)),
        ],
        out_specs=pl.BlockSpec((E, TB), lambda i: (0, i)),
        out_shape=jax.ShapeDtypeStruct((E, T), jnp.float32),
    )(x, Wr, br.reshape(1, E))

    wt, imp = _route_sc(lt)

    sel = jnp.repeat(jnp.eye(E, dtype=jnp.float32), F, axis=1)
    sel = jnp.pad(sel, ((0, PW - E), (0, 0)))
    out, loss = pl.pallas_call(
        functools.partial(_moe_kernel, num_tiles=num_tiles),
        grid=(num_tiles,),
        in_specs=[
            pl.BlockSpec((TB, D), lambda i: (i, 0)),
            pl.BlockSpec((E, TB), lambda i: (0, i)),
            pl.BlockSpec((NW, E, L), lambda i: (0, 0, 0)),
            pl.BlockSpec((E, D, F), lambda i: (0, 0, 0)),
            pl.BlockSpec((1, EF), lambda i: (0, 0)),
            pl.BlockSpec((EF, D), lambda i: (0, 0)),
            pl.BlockSpec((E, D), lambda i: (0, 0)),
            pl.BlockSpec((PW, EF), lambda i: (0, 0)),
        ],
        out_specs=[
            pl.BlockSpec((TB, D), lambda i: (i, 0)),
            pl.BlockSpec((1, 1), lambda i: (0, 0)),
        ],
        out_shape=[
            jax.ShapeDtypeStruct((T, D), jnp.float32),
            jax.ShapeDtypeStruct((1, 1), jnp.float32),
        ],
        scratch_shapes=[
            pltpu.VMEM((D, EF), jnp.bfloat16),
            pltpu.VMEM((EF + PW, D), jnp.bfloat16),
        ],
        compiler_params=pltpu.CompilerParams(
            dimension_semantics=("arbitrary",),
        ),
    )(x, wt, imp, W1, b1.reshape(1, EF), W2.reshape(EF, D), b2, sel)
    return out, loss[0, 0]


# speedup vs baseline: 1.4682x; 1.4682x over previous
import functools

import jax
import jax.numpy as jnp
from jax.experimental import pallas as pl
from jax.experimental.pallas import tpu as pltpu

T = 8192
D = 768
F = 128
E = 8
TB = 512
EF = E * F
PW = 128


def _moe_kernel(x_ref, wr_ref, br_ref, w1_ref, b1_ref, w2_ref, b2_ref,
                sel_ref, out_ref, imp_ref, loss_ref, w1c_ref, w2a_ref,
                *, num_tiles):
    i = pl.program_id(0)

    @pl.when(i == 0)
    def _stage():
        for e_i in range(E):
            w1c_ref[:, e_i * F:(e_i + 1) * F] = (
                w1_ref[e_i].astype(jnp.bfloat16))
        w2a_ref[0:EF, :] = w2_ref[...].astype(jnp.bfloat16)
        w2a_ref[EF:EF + E, :] = b2_ref[...].astype(jnp.bfloat16)
        w2a_ref[EF + E:, :] = jnp.zeros((PW - E, D), jnp.bfloat16)
        imp_ref[...] = jnp.zeros_like(imp_ref)

    x = x_ref[...]

    logits = jnp.dot(x, wr_ref[...], preferred_element_type=jnp.float32)
    logits = logits + br_ref[...]
    lt = logits.T
    m = jnp.max(lt, axis=0, keepdims=True)
    ex = jnp.exp(lt - m)
    scores = ex / jnp.sum(ex, axis=0, keepdims=True)

    iota = jax.lax.broadcasted_iota(jnp.int32, (E, TB), 0)
    v1 = jnp.max(scores, axis=0, keepdims=True)
    idx1 = jnp.min(jnp.where(scores == v1, iota, E), axis=0, keepdims=True)
    mask1 = iota == idx1
    s2 = jnp.where(mask1, -jnp.inf, scores)
    v2 = jnp.max(s2, axis=0, keepdims=True)
    idx2 = jnp.min(jnp.where(s2 == v2, iota, E), axis=0, keepdims=True)
    wt = jnp.where(mask1 | (iota == idx2), scores, 0.0)

    imp_ref[...] += jnp.sum(wt, axis=1, keepdims=True)
    w = wt.T

    xb = x.astype(jnp.bfloat16)
    h = jnp.dot(xb, w1c_ref[...], preferred_element_type=jnp.float32)
    h = jnp.maximum(h + b1_ref[...], 0.0)

    wpad_f = jnp.pad(w, ((0, 0), (0, PW - E)))
    wexp = jnp.dot(wpad_f, sel_ref[...], preferred_element_type=jnp.float32)
    hw = (h * wexp).astype(jnp.bfloat16)
    hcat = jnp.concatenate([hw, wpad_f.astype(jnp.bfloat16)], axis=-1)
    out_ref[...] = jnp.dot(hcat, w2a_ref[...],
                           preferred_element_type=jnp.float32)

    @pl.when(i == num_tiles - 1)
    def _loss():
        imp = imp_ref[...]
        mean = jnp.sum(imp) / E
        var = jnp.sum((imp - mean) ** 2) / (E - 1)
        loss_ref[...] = (var / (mean * mean + 1e-9)).reshape(1, 1)


def kernel(x, Wr, br, W1, b1, W2, b2):
    num_tiles = T // TB
    sel = jnp.repeat(jnp.eye(E, dtype=jnp.float32), F, axis=1)
    sel = jnp.pad(sel, ((0, PW - E), (0, 0)))
    out, imp, loss = pl.pallas_call(
        functools.partial(_moe_kernel, num_tiles=num_tiles),
        grid=(num_tiles,),
        in_specs=[
            pl.BlockSpec((TB, D), lambda i: (i, 0)),
            pl.BlockSpec((D, E), lambda i: (0, 0)),
            pl.BlockSpec((1, E), lambda i: (0, 0)),
            pl.BlockSpec((E, D, F), lambda i: (0, 0, 0)),
            pl.BlockSpec((1, EF), lambda i: (0, 0)),
            pl.BlockSpec((EF, D), lambda i: (0, 0)),
            pl.BlockSpec((E, D), lambda i: (0, 0)),
            pl.BlockSpec((PW, EF), lambda i: (0, 0)),
        ],
        out_specs=[
            pl.BlockSpec((TB, D), lambda i: (i, 0)),
            pl.BlockSpec((E, 1), lambda i: (0, 0)),
            pl.BlockSpec((1, 1), lambda i: (0, 0)),
        ],
        out_shape=[
            jax.ShapeDtypeStruct((T, D), jnp.float32),
            jax.ShapeDtypeStruct((E, 1), jnp.float32),
            jax.ShapeDtypeStruct((1, 1), jnp.float32),
        ],
        scratch_shapes=[
            pltpu.VMEM((D, EF), jnp.bfloat16),
            pltpu.VMEM((EF + PW, D), jnp.bfloat16),
        ],
        compiler_params=pltpu.CompilerParams(
            dimension_semantics=("arbitrary",),
        ),
    )(x, Wr, br.reshape(1, E), W1, b1.reshape(1, EF),
      W2.reshape(EF, D), b2, sel)
    del imp
    return out, loss[0, 0]


# speedup vs baseline: 1.5621x; 1.0639x over previous
import functools

import jax
import jax.numpy as jnp
from jax.experimental import pallas as pl
from jax.experimental.pallas import tpu as pltpu

T = 8192
D = 768
F = 128
E = 8
TB = 1024
EF = E * F
PW = 128


def _moe_kernel(x_ref, wr_ref, br_ref, w1_ref, b1_ref, w2_ref, b2_ref,
                sel_ref, out_ref, imp_ref, loss_ref, w1c_ref, w2a_ref,
                *, num_tiles):
    i = pl.program_id(0)

    @pl.when(i == 0)
    def _stage():
        for e_i in range(E):
            w1c_ref[:, e_i * F:(e_i + 1) * F] = (
                w1_ref[e_i].astype(jnp.bfloat16))
        w2a_ref[0:EF, :] = w2_ref[...].astype(jnp.bfloat16)
        w2a_ref[EF:EF + E, :] = b2_ref[...].astype(jnp.bfloat16)
        w2a_ref[EF + E:, :] = jnp.zeros((PW - E, D), jnp.bfloat16)
        imp_ref[...] = jnp.zeros_like(imp_ref)

    x = x_ref[...]

    logits = jnp.dot(x, wr_ref[...], preferred_element_type=jnp.float32)
    logits = logits + br_ref[...]
    lt = logits.T
    m = jnp.max(lt, axis=0, keepdims=True)
    ex = jnp.exp(lt - m)
    scores = ex / jnp.sum(ex, axis=0, keepdims=True)

    iota = jax.lax.broadcasted_iota(jnp.int32, (E, TB), 0)
    v1 = jnp.max(scores, axis=0, keepdims=True)
    idx1 = jnp.min(jnp.where(scores == v1, iota, E), axis=0, keepdims=True)
    mask1 = iota == idx1
    s2 = jnp.where(mask1, -jnp.inf, scores)
    v2 = jnp.max(s2, axis=0, keepdims=True)
    idx2 = jnp.min(jnp.where(s2 == v2, iota, E), axis=0, keepdims=True)
    wt = jnp.where(mask1 | (iota == idx2), scores, 0.0)

    imp_ref[...] += jnp.sum(wt, axis=1, keepdims=True)
    w = wt.T

    xb = x.astype(jnp.bfloat16)
    h = jnp.dot(xb, w1c_ref[...], preferred_element_type=jnp.float32)
    h = jnp.maximum(h + b1_ref[...], 0.0)

    wpad_f = jnp.pad(w, ((0, 0), (0, PW - E)))
    wexp = jnp.dot(wpad_f, sel_ref[...], preferred_element_type=jnp.float32)
    hw = (h * wexp).astype(jnp.bfloat16)
    hcat = jnp.concatenate([hw, wpad_f.astype(jnp.bfloat16)], axis=-1)
    out_ref[...] = jnp.dot(hcat, w2a_ref[...],
                           preferred_element_type=jnp.float32)

    @pl.when(i == num_tiles - 1)
    def _loss():
        imp = imp_ref[...]
        mean = jnp.sum(imp) / E
        var = jnp.sum((imp - mean) ** 2) / (E - 1)
        loss_ref[...] = (var / (mean * mean + 1e-9)).reshape(1, 1)


def kernel(x, Wr, br, W1, b1, W2, b2):
    num_tiles = T // TB
    sel = jnp.repeat(jnp.eye(E, dtype=jnp.float32), F, axis=1)
    sel = jnp.pad(sel, ((0, PW - E), (0, 0)))
    out, imp, loss = pl.pallas_call(
        functools.partial(_moe_kernel, num_tiles=num_tiles),
        grid=(num_tiles,),
        in_specs=[
            pl.BlockSpec((TB, D), lambda i: (i, 0)),
            pl.BlockSpec((D, E), lambda i: (0, 0)),
            pl.BlockSpec((1, E), lambda i: (0, 0)),
            pl.BlockSpec((E, D, F), lambda i: (0, 0, 0)),
            pl.BlockSpec((1, EF), lambda i: (0, 0)),
            pl.BlockSpec((EF, D), lambda i: (0, 0)),
            pl.BlockSpec((E, D), lambda i: (0, 0)),
            pl.BlockSpec((PW, EF), lambda i: (0, 0)),
        ],
        out_specs=[
            pl.BlockSpec((TB, D), lambda i: (i, 0)),
            pl.BlockSpec((E, 1), lambda i: (0, 0)),
            pl.BlockSpec((1, 1), lambda i: (0, 0)),
        ],
        out_shape=[
            jax.ShapeDtypeStruct((T, D), jnp.float32),
            jax.ShapeDtypeStruct((E, 1), jnp.float32),
            jax.ShapeDtypeStruct((1, 1), jnp.float32),
        ],
        scratch_shapes=[
            pltpu.VMEM((D, EF), jnp.bfloat16),
            pltpu.VMEM((EF + PW, D), jnp.bfloat16),
        ],
        compiler_params=pltpu.CompilerParams(
            dimension_semantics=("arbitrary",),
        ),
    )(x, Wr, br.reshape(1, E), W1, b1.reshape(1, EF),
      W2.reshape(EF, D), b2, sel)
    del imp
    return out, loss[0, 0]
